# baseline (speedup 1.0000x reference)
import jax
import jax.numpy as jnp
from jax import lax
from jax.experimental import pallas as pl
from jax.experimental.pallas import tpu as pltpu
from jax.experimental.pallas import tpu_sc as plsc

B = 16384
D = 128
L = 16
NC = 2
NS = 16
NW = NC * NS
B_PER_W = B // NW

SIZES = [128, 128, 128, 64, 64]
OFFS = [sum(SIZES[:i]) for i in range(len(SIZES))]
NCHUNK = len(SIZES)
BUF_OF = [0, 1, 0, 2, 3]
BUF_ROWS = [128, 128, 64, 64]
PRIME = [0, 1, 3, 4]
NEXT_START = {0: [2]}


def _body(uidx_hbm, iidx_hbm, utab_hbm, itab_hbm, out_hbm, *scr):
  nb = len(BUF_ROWS)
  uidx_v, iidx_v = scr[0], scr[1]
  ubufs = list(scr[2:2 + nb])
  ibufs = list(scr[2 + nb:2 + 2 * nb])
  part, out_v = scr[2 + 2 * nb], scr[3 + 2 * nb]
  sus = list(scr[4 + 2 * nb:4 + 3 * nb])
  sis = list(scr[4 + 3 * nb:4 + 4 * nb])
  sx = scr[4 + 4 * nb]

  wid = lax.axis_index("s") * NC + lax.axis_index("c")
  base = wid * B_PER_W

  stage = []
  for c in range(NCHUNK):
    stage.append((
        pltpu.async_copy(
            uidx_hbm.at[pl.ds(base + OFFS[c], SIZES[c])],
            uidx_v.at[c, pl.ds(0, SIZES[c])], sx),
        pltpu.async_copy(
            iidx_hbm.at[pl.ds(base + OFFS[c], SIZES[c])],
            iidx_v.at[c, pl.ds(0, SIZES[c])], sx),
    ))

  iot = lax.iota(jnp.int32, L)

  def start(c):
    p = BUF_OF[c]
    stage[c][0].wait()
    stage[c][1].wait()
    cu = pltpu.async_copy(
        utab_hbm.at[uidx_v.at[c, pl.ds(0, SIZES[c])]], ubufs[p], sus[p])
    ci = pltpu.async_copy(
        itab_hbm.at[iidx_v.at[c, pl.ds(0, SIZES[c])]], ibufs[p], sis[p])
    return cu, ci

  pend = {}
  for c in PRIME:
    pend[c] = start(c)

  outcp = []
  for c in range(NCHUNK):
    p = BUF_OF[c]
    pend[c][0].wait()
    pend[c][1].wait()
    ur, ir = ubufs[p], ibufs[p]

    def group_body(g, carry, ur=ur, ir=ir, c=c):
      for r in range(L):
        row = g * L + r
        acc = ur[row, 0:L] * ir[row, 0:L]
        for k in range(1, D // L):
          acc = acc + ur[row, k * L:(k + 1) * L] * ir[row, k * L:(k + 1) * L]
        part[pl.ds(r * (L + 1), L)] = acc
      res = plsc.load_gather(part, [iot * (L + 1)])
      for cc in range(1, L):
        res = res + plsc.load_gather(part, [iot * (L + 1) + cc])
      out_v[pl.ds(OFFS[c] + g * L, L)] = res
      return carry

    lax.fori_loop(0, SIZES[c] // L, group_body, 0)
    for nxt in NEXT_START.get(c, []):
      pend[nxt] = start(nxt)
    outcp.append(pltpu.async_copy(
        out_v.at[pl.ds(OFFS[c], SIZES[c])],
        out_hbm.at[pl.ds(base + OFFS[c], SIZES[c])], sx))

  for cp in outcp:
    cp.wait()


@jax.jit
def _run(user_indices, item_indices, user_table, item_table):
  mesh = plsc.VectorSubcoreMesh(core_axis_name="c", subcore_axis_name="s")
  f = pl.kernel(
      _body,
      out_type=jax.ShapeDtypeStruct((B,), jnp.float32),
      mesh=mesh,
      compiler_params=pltpu.CompilerParams(needs_layout_passes=False),
      scratch_types=(
          [pltpu.VMEM((NCHUNK, 128), jnp.int32)] * 2
          + [pltpu.VMEM((r, D), jnp.float32) for r in BUF_ROWS] * 2
          + [pltpu.VMEM((L * (L + 1),), jnp.float32),
             pltpu.VMEM((B_PER_W,), jnp.float32)]
          + [pltpu.SemaphoreType.DMA] * (2 * len(BUF_ROWS) + 1)
      ),
  )
  return f(user_indices, item_indices, user_table, item_table)


def kernel(user_indices, item_indices, user_table, item_table):
  return _run(user_indices.astype(jnp.int32), item_indices.astype(jnp.int32),
              user_table, item_table)


# speedup vs baseline: 1.0006x; 1.0006x over previous
import jax
import jax.numpy as jnp
from jax import lax
from jax.experimental import pallas as pl
from jax.experimental.pallas import tpu as pltpu
from jax.experimental.pallas import tpu_sc as plsc

B = 16384
D = 128
L = 16
NC = 2
NS = 16
NW = NC * NS
B_PER_W = B // NW

SIZES = [128, 128, 128, 96, 32]
OFFS = [sum(SIZES[:i]) for i in range(len(SIZES))]
NCHUNK = len(SIZES)
BUF_OF = [0, 1, 2, 0, 3]
BUF_ROWS = [128, 128, 128, 32]
PRIME = [0, 1, 2, 4]
NEXT_START = {0: [3]}


def _body(uidx_hbm, iidx_hbm, utab_hbm, itab_hbm, out_hbm, *scr):
  nb = len(BUF_ROWS)
  uidx_v, iidx_v = scr[0], scr[1]
  ubufs = list(scr[2:2 + nb])
  ibufs = list(scr[2 + nb:2 + 2 * nb])
  part, out_v = scr[2 + 2 * nb], scr[3 + 2 * nb]
  sus = list(scr[4 + 2 * nb:4 + 3 * nb])
  sis = list(scr[4 + 3 * nb:4 + 4 * nb])
  sx = scr[4 + 4 * nb]

  wid = lax.axis_index("s") * NC + lax.axis_index("c")
  base = wid * B_PER_W

  stage = []
  for c in range(NCHUNK):
    stage.append((
        pltpu.async_copy(
            uidx_hbm.at[pl.ds(base + OFFS[c], SIZES[c])],
            uidx_v.at[c, pl.ds(0, SIZES[c])], sx),
        pltpu.async_copy(
            iidx_hbm.at[pl.ds(base + OFFS[c], SIZES[c])],
            iidx_v.at[c, pl.ds(0, SIZES[c])], sx),
    ))

  iot = lax.iota(jnp.int32, L)

  def start(c):
    p = BUF_OF[c]
    stage[c][0].wait()
    stage[c][1].wait()
    udst = ubufs[p] if SIZES[c] == BUF_ROWS[p] else ubufs[p].at[pl.ds(0, SIZES[c])]
    idst = ibufs[p] if SIZES[c] == BUF_ROWS[p] else ibufs[p].at[pl.ds(0, SIZES[c])]
    cu = pltpu.async_copy(
        utab_hbm.at[uidx_v.at[c, pl.ds(0, SIZES[c])]], udst, sus[p])
    ci = pltpu.async_copy(
        itab_hbm.at[iidx_v.at[c, pl.ds(0, SIZES[c])]], idst, sis[p])
    return cu, ci

  pend = {}
  for c in PRIME:
    pend[c] = start(c)

  outcp = []
  for c in range(NCHUNK):
    p = BUF_OF[c]
    pend[c][0].wait()
    pend[c][1].wait()
    ur, ir = ubufs[p], ibufs[p]

    def group_body(g, carry, ur=ur, ir=ir, c=c):
      for r in range(L):
        row = g * L + r
        acc = ur[row, 0:L] * ir[row, 0:L]
        for k in range(1, D // L):
          acc = acc + ur[row, k * L:(k + 1) * L] * ir[row, k * L:(k + 1) * L]
        part[pl.ds(r * (L + 1), L)] = acc
      res = plsc.load_gather(part, [iot * (L + 1)])
      for cc in range(1, L):
        res = res + plsc.load_gather(part, [iot * (L + 1) + cc])
      out_v[pl.ds(OFFS[c] + g * L, L)] = res
      return carry

    lax.fori_loop(0, SIZES[c] // L, group_body, 0)
    for nxt in NEXT_START.get(c, []):
      pend[nxt] = start(nxt)
    outcp.append(pltpu.async_copy(
        out_v.at[pl.ds(OFFS[c], SIZES[c])],
        out_hbm.at[pl.ds(base + OFFS[c], SIZES[c])], sx))

  for cp in outcp:
    cp.wait()


@jax.jit
def _run(user_indices, item_indices, user_table, item_table):
  mesh = plsc.VectorSubcoreMesh(core_axis_name="c", subcore_axis_name="s")
  f = pl.kernel(
      _body,
      out_type=jax.ShapeDtypeStruct((B,), jnp.float32),
      mesh=mesh,
      compiler_params=pltpu.CompilerParams(needs_layout_passes=False),
      scratch_types=(
          [pltpu.VMEM((NCHUNK, 128), jnp.int32)] * 2
          + [pltpu.VMEM((r, D), jnp.float32) for r in BUF_ROWS] * 2
          + [pltpu.VMEM((L * (L + 1),), jnp.float32),
             pltpu.VMEM((B_PER_W,), jnp.float32)]
          + [pltpu.SemaphoreType.DMA] * (2 * len(BUF_ROWS) + 1)
      ),
  )
  return f(user_indices, item_indices, user_table, item_table)


def kernel(user_indices, item_indices, user_table, item_table):
  return _run(user_indices.astype(jnp.int32), item_indices.astype(jnp.int32),
              user_table, item_table)
